# speedup vs baseline: 6.6654x; 6.6654x over previous
import functools

import jax
import jax.numpy as jnp
from jax import lax
from jax.experimental import pallas as pl
from jax.experimental.pallas import tpu as pltpu
from jax.experimental.pallas import tpu_sc as plsc

N_NODES = 10000
D = 128
H = 256
E_EDGES = 320000
NEXP = 4

NW = 32
ROWS_TOTAL = 2 * E_EDGES
RPW = ROWS_TOTAL // NW
CHUNK = 80
NCHUNK = RPW // CHUNK
KBUF = 5

BE = 2560
NB = E_EDGES // BE
WCOL = 640


def _sc_gather(z, idx3):
    mesh = plsc.VectorSubcoreMesh(core_axis_name="c", subcore_axis_name="s")

    @functools.partial(
        pl.kernel,
        mesh=mesh,
        out_type=jax.ShapeDtypeStruct((ROWS_TOTAL, D), jnp.float32),
        scratch_types=(
            [pltpu.VMEM((NCHUNK, CHUNK), jnp.int32)]
            + [pltpu.VMEM((CHUNK, D), jnp.float32) for _ in range(KBUF)]
            + [pltpu.SemaphoreType.DMA for _ in range(2 * KBUF)]
        ),
    )
    def gather_kernel(z_hbm, idx_hbm, out_hbm, idx_v, *rest):
        rows = rest[:KBUF]
        gsem = rest[KBUF:2 * KBUF]
        osem = rest[2 * KBUF:]
        wid = lax.axis_index("s") * 2 + lax.axis_index("c")
        base = wid * RPW
        pltpu.sync_copy(idx_hbm.at[wid], idx_v)

        def outer(j, _):
            c0 = j * KBUF
            handles = []
            for s in range(KBUF):
                @pl.when(j > 0)
                def _wait_out(s=s):
                    pltpu.make_async_copy(
                        rows[s], out_hbm.at[pl.ds(0, CHUNK)], osem[s]
                    ).wait()
                handles.append(
                    pltpu.async_copy(z_hbm.at[idx_v.at[c0 + s]], rows[s], gsem[s])
                )
            for s in range(KBUF):
                handles[s].wait()
                pltpu.async_copy(
                    rows[s],
                    out_hbm.at[pl.ds(base + (c0 + s) * CHUNK, CHUNK)],
                    osem[s],
                )
            return ()

        lax.fori_loop(0, NCHUNK // KBUF, outer, (), unroll=False)
        for s in range(KBUF):
            pltpu.make_async_copy(
                rows[s], out_hbm.at[pl.ds(0, CHUNK)], osem[s]
            ).wait()

    return gather_kernel(z, idx3)


def _tc_body(zu_ref, zv_ref, w1_ref, b1_ref, gb_ref, w2_ref, b2_ref,
             out_ref, aux_ref, acc_ref):
    i = pl.program_id(0)
    zu = zu_ref[...]
    zv = zv_ref[...]
    dd = jnp.abs(zu - zv)
    mm = zu * zv
    Tu = jnp.dot(zu, w1_ref[0], preferred_element_type=jnp.float32)
    Tv = jnp.dot(zv, w1_ref[1], preferred_element_type=jnp.float32)
    Td = jnp.dot(dd, w1_ref[2], preferred_element_type=jnp.float32)
    Tm = jnp.dot(mm, w1_ref[3], preferred_element_type=jnp.float32)
    b1 = b1_ref[...]
    h1 = jnp.maximum(Tu[:, 0:H] + Tv[:, 0:H] + b1[0], 0.0)
    h2 = jnp.maximum(Td[:, 0:H] + b1[1], 0.0)
    h3 = jnp.maximum(Tm[:, 0:H] + b1[2], 0.0)
    h4 = jnp.maximum(
        Tu[:, H:2 * H] + Tv[:, H:2 * H] + Td[:, H:2 * H] + Tm[:, H:2 * H]
        + b1[3], 0.0)
    gl = (Tu[:, 2 * H:WCOL] + Tv[:, 2 * H:WCOL] + Td[:, 2 * H:WCOL]
          + Tm[:, 2 * H:WCOL] + gb_ref[...])
    gmax = jnp.max(gl, axis=1, keepdims=True)
    ge = jnp.exp(gl - gmax)
    gp = ge / jnp.sum(ge, axis=1, keepdims=True)
    w2 = w2_ref[...]
    b2 = b2_ref[...]
    s1 = jnp.sum(h1 * w2[0], axis=1, keepdims=True) + b2[0:1]
    s2 = jnp.sum(h2 * w2[1], axis=1, keepdims=True) + b2[1:2]
    s3 = jnp.sum(h3 * w2[2], axis=1, keepdims=True) + b2[2:3]
    s4 = jnp.sum(h4 * w2[3], axis=1, keepdims=True) + b2[3:4]
    pmax = jnp.max(gp, axis=1, keepdims=True)
    lane = lax.broadcasted_iota(jnp.int32, (BE, 128), 1)
    amin = jnp.min(jnp.where(gp >= pmax, lane, 128), axis=1, keepdims=True)
    sel = jnp.where(amin == 0, s1,
                    jnp.where(amin == 1, s2,
                              jnp.where(amin == 2, s3, s4)))
    out_ref[...] = pmax * sel

    @pl.when(i == 0)
    def _init():
        acc_ref[...] = jnp.zeros_like(acc_ref)

    acc_ref[...] += jnp.sum(gp, axis=0, keepdims=True)

    @pl.when(i == NB - 1)
    def _finish():
        avg = acc_ref[...] / float(E_EDGES)
        aux_ref[...] = (jnp.sum(avg * avg) * float(NEXP)).reshape(1, 1)


def _tc_moe(gathered, W1p, b1p, gbp, W2p, b2p):
    return pl.pallas_call(
        _tc_body,
        grid=(NB,),
        in_specs=[
            pl.BlockSpec((BE, D), lambda i: (i, 0)),
            pl.BlockSpec((BE, D), lambda i: (i + NB, 0)),
            pl.BlockSpec((NEXP, D, WCOL), lambda i: (0, 0, 0)),
            pl.BlockSpec((NEXP, H), lambda i: (0, 0)),
            pl.BlockSpec((1, 128), lambda i: (0, 0)),
            pl.BlockSpec((NEXP, H), lambda i: (0, 0)),
            pl.BlockSpec((NEXP, 1), lambda i: (0, 0)),
        ],
        out_specs=[
            pl.BlockSpec((BE, 1), lambda i: (i, 0)),
            pl.BlockSpec((1, 1), lambda i: (0, 0)),
        ],
        out_shape=[
            jax.ShapeDtypeStruct((E_EDGES, 1), jnp.float32),
            jax.ShapeDtypeStruct((1, 1), jnp.float32),
        ],
        scratch_shapes=[pltpu.VMEM((1, 128), jnp.float32)],
    )(gathered, gathered, W1p, b1p, gbp, W2p, b2p)


def kernel(g, z, uv, gate_W, gate_b, ec_W1, ec_b1, ec_W2, ec_b2,
           ed_W1, ed_b1, ed_W2, ed_b2, em_W1, em_b1, em_W2, em_b2,
           ea_W1, ea_b1, ea_W2, ea_b2):
    idx3 = uv.reshape(NW, NCHUNK, CHUNK)
    gathered = _sc_gather(z, idx3)

    pad = jnp.zeros((D, WCOL - 2 * H - NEXP), jnp.float32)
    Pu = jnp.concatenate([ec_W1[:D], ea_W1[:D], gate_W[:D], pad], axis=1)
    Pv = jnp.concatenate([ec_W1[D:], ea_W1[D:2 * D], gate_W[D:2 * D], pad], axis=1)
    Pd = jnp.concatenate([ed_W1, ea_W1[2 * D:3 * D], gate_W[2 * D:3 * D], pad], axis=1)
    Pm = jnp.concatenate([em_W1, ea_W1[3 * D:], gate_W[3 * D:], pad], axis=1)
    W1p = jnp.stack([Pu, Pv, Pd, Pm])
    b1p = jnp.stack([ec_b1, ed_b1, em_b1, ea_b1])
    gbp = jnp.concatenate(
        [gate_b, jnp.full((128 - NEXP,), -1e30, jnp.float32)])[None, :]
    W2p = jnp.stack([ec_W2[:, 0], ed_W2[:, 0], em_W2[:, 0], ea_W2[:, 0]])
    b2p = jnp.stack([ec_b2, ed_b2, em_b2, ea_b2])

    out, aux = _tc_moe(gathered, W1p, b1p, gbp, W2p, b2p)
    return out, aux[0, 0]


# speedup vs baseline: 6.7112x; 1.0069x over previous
import functools

import jax
import jax.numpy as jnp
from jax import lax
from jax.experimental import pallas as pl
from jax.experimental.pallas import tpu as pltpu
from jax.experimental.pallas import tpu_sc as plsc

N_NODES = 10000
D = 128
H = 256
E_EDGES = 320000
NEXP = 4

NW = 32
ROWS_TOTAL = 2 * E_EDGES
RPW = ROWS_TOTAL // NW
CHUNK = 80
NCHUNK = RPW // CHUNK
KBUF = 5

BE = 2560
NB = E_EDGES // BE
WCOL = 640


def _sc_gather(z, idx3):
    mesh = plsc.VectorSubcoreMesh(core_axis_name="c", subcore_axis_name="s")

    @functools.partial(
        pl.kernel,
        mesh=mesh,
        out_type=jax.ShapeDtypeStruct((ROWS_TOTAL, D), jnp.float32),
        scratch_types=(
            [pltpu.VMEM((NCHUNK, CHUNK), jnp.int32)]
            + [pltpu.VMEM((CHUNK, D), jnp.float32) for _ in range(KBUF)]
            + [pltpu.SemaphoreType.DMA for _ in range(2 * KBUF)]
        ),
    )
    def gather_kernel(z_hbm, idx_hbm, out_hbm, idx_v, *rest):
        rows = rest[:KBUF]
        gsem = rest[KBUF:2 * KBUF]
        osem = rest[2 * KBUF:]
        wid = lax.axis_index("s") * 2 + lax.axis_index("c")
        base = wid * RPW
        pltpu.sync_copy(idx_hbm.at[wid], idx_v)

        def outer(j, _):
            c0 = j * KBUF
            handles = []
            for s in range(KBUF):
                @pl.when(j > 0)
                def _wait_out(s=s):
                    pltpu.make_async_copy(
                        rows[s], out_hbm.at[pl.ds(0, CHUNK)], osem[s]
                    ).wait()
                handles.append(
                    pltpu.async_copy(z_hbm.at[idx_v.at[c0 + s]], rows[s], gsem[s])
                )
            for s in range(KBUF):
                handles[s].wait()
                pltpu.async_copy(
                    rows[s],
                    out_hbm.at[pl.ds(base + (c0 + s) * CHUNK, CHUNK)],
                    osem[s],
                )
            return ()

        lax.fori_loop(0, NCHUNK // KBUF, outer, (), unroll=False)
        for s in range(KBUF):
            pltpu.make_async_copy(
                rows[s], out_hbm.at[pl.ds(0, CHUNK)], osem[s]
            ).wait()

    return gather_kernel(z, idx3)


def _tc_body(zu_ref, zv_ref, w1_ref, g_ref, b1_ref, gb_ref, w2_ref, b2_ref,
             out_ref, aux_ref, acc_ref):
    i = pl.program_id(0)
    zu = zu_ref[...]
    zv = zv_ref[...]
    dd = jnp.abs(zu - zv)
    mm = zu * zv
    zub = zu.astype(jnp.bfloat16)
    zvb = zv.astype(jnp.bfloat16)
    ddb = dd.astype(jnp.bfloat16)
    mmb = mm.astype(jnp.bfloat16)
    Tu = jnp.dot(zub, w1_ref[0], preferred_element_type=jnp.float32)
    Tv = jnp.dot(zvb, w1_ref[1], preferred_element_type=jnp.float32)
    Td = jnp.dot(ddb, w1_ref[2], preferred_element_type=jnp.float32)
    Tm = jnp.dot(mmb, w1_ref[3], preferred_element_type=jnp.float32)
    b1 = b1_ref[...]
    h1 = jnp.maximum(Tu[:, 0:H] + Tv[:, 0:H] + b1[0], 0.0)
    h2 = jnp.maximum(Td[:, 0:H] + b1[1], 0.0)
    h3 = jnp.maximum(Tm[:, 0:H] + b1[2], 0.0)
    h4 = jnp.maximum(
        Tu[:, H:2 * H] + Tv[:, H:2 * H] + Td[:, H:2 * H] + Tm[:, H:2 * H]
        + b1[3], 0.0)
    gl = (jnp.dot(zu, g_ref[0], preferred_element_type=jnp.float32)
          + jnp.dot(zv, g_ref[1], preferred_element_type=jnp.float32)
          + jnp.dot(dd, g_ref[2], preferred_element_type=jnp.float32)
          + jnp.dot(mm, g_ref[3], preferred_element_type=jnp.float32)
          + gb_ref[...])
    gmax = jnp.max(gl, axis=1, keepdims=True)
    ge = jnp.exp(gl - gmax)
    gp = ge / jnp.sum(ge, axis=1, keepdims=True)
    w2 = w2_ref[...]
    b2 = b2_ref[...]
    s1 = jnp.sum(h1 * w2[0], axis=1, keepdims=True) + b2[0:1]
    s2 = jnp.sum(h2 * w2[1], axis=1, keepdims=True) + b2[1:2]
    s3 = jnp.sum(h3 * w2[2], axis=1, keepdims=True) + b2[2:3]
    s4 = jnp.sum(h4 * w2[3], axis=1, keepdims=True) + b2[3:4]
    pmax = jnp.max(gp, axis=1, keepdims=True)
    lane = lax.broadcasted_iota(jnp.int32, (BE, 128), 1)
    amin = jnp.min(jnp.where(gp >= pmax, lane, 128), axis=1, keepdims=True)
    sel = jnp.where(amin == 0, s1,
                    jnp.where(amin == 1, s2,
                              jnp.where(amin == 2, s3, s4)))
    out_ref[...] = pmax * sel

    @pl.when(i == 0)
    def _init():
        acc_ref[...] = jnp.zeros_like(acc_ref)

    acc_ref[...] += jnp.sum(gp, axis=0, keepdims=True)

    @pl.when(i == NB - 1)
    def _finish():
        avg = acc_ref[...] / float(E_EDGES)
        aux_ref[...] = (jnp.sum(avg * avg) * float(NEXP)).reshape(1, 1)


def _tc_moe(gathered, W1p, Gp, b1p, gbp, W2p, b2p):
    return pl.pallas_call(
        _tc_body,
        grid=(NB,),
        in_specs=[
            pl.BlockSpec((BE, D), lambda i: (i, 0)),
            pl.BlockSpec((BE, D), lambda i: (i + NB, 0)),
            pl.BlockSpec((NEXP, D, 2 * H), lambda i: (0, 0, 0)),
            pl.BlockSpec((NEXP, D, 128), lambda i: (0, 0, 0)),
            pl.BlockSpec((NEXP, H), lambda i: (0, 0)),
            pl.BlockSpec((1, 128), lambda i: (0, 0)),
            pl.BlockSpec((NEXP, H), lambda i: (0, 0)),
            pl.BlockSpec((NEXP, 1), lambda i: (0, 0)),
        ],
        out_specs=[
            pl.BlockSpec((BE, 1), lambda i: (i, 0)),
            pl.BlockSpec((1, 1), lambda i: (0, 0)),
        ],
        out_shape=[
            jax.ShapeDtypeStruct((E_EDGES, 1), jnp.float32),
            jax.ShapeDtypeStruct((1, 1), jnp.float32),
        ],
        scratch_shapes=[pltpu.VMEM((1, 128), jnp.float32)],
    )(gathered, gathered, W1p, Gp, b1p, gbp, W2p, b2p)


def kernel(g, z, uv, gate_W, gate_b, ec_W1, ec_b1, ec_W2, ec_b2,
           ed_W1, ed_b1, ed_W2, ed_b2, em_W1, em_b1, em_W2, em_b2,
           ea_W1, ea_b1, ea_W2, ea_b2):
    idx3 = uv.reshape(NW, NCHUNK, CHUNK)
    gathered = _sc_gather(z, idx3)

    Pu = jnp.concatenate([ec_W1[:D], ea_W1[:D]], axis=1)
    Pv = jnp.concatenate([ec_W1[D:], ea_W1[D:2 * D]], axis=1)
    Pd = jnp.concatenate([ed_W1, ea_W1[2 * D:3 * D]], axis=1)
    Pm = jnp.concatenate([em_W1, ea_W1[3 * D:]], axis=1)
    W1p = jnp.stack([Pu, Pv, Pd, Pm]).astype(jnp.bfloat16)
    gpad = jnp.zeros((D, 128 - NEXP), jnp.float32)
    Gp = jnp.stack([
        jnp.concatenate([gate_W[:D], gpad], axis=1),
        jnp.concatenate([gate_W[D:2 * D], gpad], axis=1),
        jnp.concatenate([gate_W[2 * D:3 * D], gpad], axis=1),
        jnp.concatenate([gate_W[3 * D:], gpad], axis=1),
    ])
    b1p = jnp.stack([ec_b1, ed_b1, em_b1, ea_b1])
    gbp = jnp.concatenate(
        [gate_b, jnp.full((128 - NEXP,), -1e30, jnp.float32)])[None, :]
    W2p = jnp.stack([ec_W2[:, 0], ed_W2[:, 0], em_W2[:, 0], ea_W2[:, 0]])
    b2p = jnp.stack([ec_b2, ed_b2, em_b2, ea_b2])

    out, aux = _tc_moe(gathered, W1p, Gp, b1p, gbp, W2p, b2p)
    return out, aux[0, 0]
